# baseline (speedup 1.0000x reference)
import functools

import jax
import jax.numpy as jnp
from jax.experimental import pallas as pl

_B, _N, _GH, _GW = 32, 256, 16, 16
_P = _GH * _GW
_D_EDGE, _HID = 16, 128
_WH = 32
_RC = 256


def _edge_kernel(relt_ref, adj_ref, w0t_ref, b0_ref, w1t_ref, b1_ref,
                 w2t_ref, b2_ref, out_ref):
    relt = relt_ref[0]
    dn = (((2,), (1,)), ((0,), (0,)))
    h = jax.lax.dot_general(
        jnp.broadcast_to(w0t_ref[...], (_RC, 4 * _WH, _D_EDGE)), relt, dn,
        preferred_element_type=jnp.float32)
    h = jnp.maximum(h + b0_ref[...], 0.0)
    h = jax.lax.dot_general(
        jnp.broadcast_to(w1t_ref[...], (_RC, 4 * _WH, 4 * _WH)), h, dn,
        preferred_element_type=jnp.float32)
    h = jnp.maximum(h + b1_ref[...], 0.0)
    logits = jax.lax.dot_general(
        jnp.broadcast_to(w2t_ref[...], (_RC, 4, 4 * _WH)), h, dn,
        preferred_element_type=jnp.float32)
    w = jax.nn.sigmoid(logits + b2_ref[...])
    adj = adj_ref[0]
    out_ref[0] = w * adj[:, None, :]


_NS = 4


def _net_kernel(x0_ref, a_ref, obs_ref,
                wn_ref, ws_ref, bd_ref,
                wc1_ref, bc1_ref,
                wsgp_ref, wkgp_ref, bp_ref,
                wsg2_ref, wkg2_ref, b2_ref,
                wm1_ref, bm1_ref, wm2_ref, bm2_ref,
                out_ref):
    node_iota = jax.lax.broadcasted_iota(jnp.int32, (_N, _P), 0)

    for s in range(_NS):
        x = x0_ref[s]
        A = a_ref[s]

        def dg(x, l):
            Al = A[:, l, :]
            deg = jnp.sum(Al, axis=1, keepdims=True) + 1e-6
            msg = (Al @ x) / deg
            return jnp.maximum(msg @ wn_ref[l] + x @ ws_ref[l] + bd_ref[l], 0.0)

        x = dg(x, 0)
        x = dg(x, 1)

        ohT = (node_iota == obs_ref[s]).astype(jnp.float32)
        dtl = (((0,), (0,)), ((), ()))

        sg0 = jax.lax.dot_general(ohT, x, dtl,
                                  preferred_element_type=jnp.float32)
        sg = jnp.maximum(sg0 @ wc1_ref[...] + bc1_ref[...], 0.0)

        counts = jnp.sum(ohT, axis=1, keepdims=True)
        pooled = (ohT @ sg) / (counts + 1e-6)
        x = jnp.maximum(pooled @ wsgp_ref[...] + x @ wkgp_ref[...] + bp_ref[...] + x, 0.0)

        x = dg(x, 2)
        x = dg(x, 3)

        g2 = jax.lax.dot_general(ohT, x, dtl,
                                 preferred_element_type=jnp.float32)
        sg = jnp.maximum(sg @ wsg2_ref[...] + g2 @ wkg2_ref[...] + b2_ref[...], 0.0)

        v = jnp.mean(sg, axis=0, keepdims=True)
        v = jnp.maximum(v @ wm1_ref[...] + bm1_ref[...], 0.0)
        v = jnp.maximum(v @ wm2_ref[...] + bm2_ref[...], 0.0)
        out_ref[s] = v


def kernel(kg_node_feats, kg_adj, kg_rel, obs, params):
    f32 = jnp.float32

    dgs = [params['dg1'], params['dg2'], params['dg3'], params['dg4']]
    w0t = jnp.concatenate([d['wnet_ws'][0] for d in dgs], axis=1).T
    b0t = jnp.concatenate([d['wnet_bs'][0] for d in dgs])[:, None]
    w1 = jnp.zeros((4 * _WH, 4 * _WH), f32)
    for i, d in enumerate(dgs):
        w1 = w1.at[i * _WH:(i + 1) * _WH, i * _WH:(i + 1) * _WH].set(d['wnet_ws'][1])
    w1t = w1.T
    b1t = jnp.concatenate([d['wnet_bs'][1] for d in dgs])[:, None]
    w2t = jnp.zeros((4, 4 * _WH), f32)
    for i, d in enumerate(dgs):
        w2t = w2t.at[i, i * _WH:(i + 1) * _WH].set(d['wnet_ws'][2][:, 0])
    b2t = jnp.stack([d['wnet_bs'][2][0] for d in dgs])[:, None]

    rel_t = jnp.swapaxes(kg_rel.astype(jnp.bfloat16), 2, 3)
    w0t = w0t.astype(jnp.bfloat16)

    grid1 = (_B, _N // _RC)
    A = pl.pallas_call(
        _edge_kernel,
        grid=grid1,
        in_specs=[
            pl.BlockSpec((1, _RC, _D_EDGE, _N), lambda b, r: (b, r, 0, 0)),
            pl.BlockSpec((1, _RC, _N), lambda b, r: (b, r, 0)),
            pl.BlockSpec((4 * _WH, _D_EDGE), lambda b, r: (0, 0)),
            pl.BlockSpec((4 * _WH, 1), lambda b, r: (0, 0)),
            pl.BlockSpec((4 * _WH, 4 * _WH), lambda b, r: (0, 0)),
            pl.BlockSpec((4 * _WH, 1), lambda b, r: (0, 0)),
            pl.BlockSpec((4, 4 * _WH), lambda b, r: (0, 0)),
            pl.BlockSpec((4, 1), lambda b, r: (0, 0)),
        ],
        out_specs=pl.BlockSpec((1, _RC, 4, _N), lambda b, r: (b, r, 0, 0)),
        out_shape=jax.ShapeDtypeStruct((_B, _N, 4, _N), f32),
    )(rel_t, kg_adj, w0t, b0t, w1t, b1t, w2t, b2t)

    wn = jnp.stack([d['w_nbr'] for d in dgs])
    ws = jnp.stack([d['w_self'] for d in dgs])
    bd = jnp.stack([d['b'] for d in dgs])
    sgc1, sgc2, sgkg = params['sgc1'], params['sgc2'], params['sgkg1']
    wc1 = sgc1['w_sg'] + sgc1['w_kg']
    bc1 = sgc1['b'][None, :]
    wsgp, wkgp, bp = sgkg['w_sg'], sgkg['w_kg'], sgkg['b'][None, :]
    wsg2, wkg2, b2s = sgc2['w_sg'], sgc2['w_kg'], sgc2['b'][None, :]
    wm1, bm1 = params['mlp']['ws'][0], params['mlp']['bs'][0][None, :]
    wm2, bm2 = params['mlp']['ws'][1], params['mlp']['bs'][1][None, :]

    obs3 = obs.reshape(_B, 1, _P)

    full = lambda shape: pl.BlockSpec(shape, lambda b: tuple(0 for _ in shape))
    out = pl.pallas_call(
        _net_kernel,
        grid=(_B // _NS,),
        in_specs=[
            pl.BlockSpec((_NS, _N, _HID), lambda b: (b, 0, 0)),
            pl.BlockSpec((_NS, _N, 4, _N), lambda b: (b, 0, 0, 0)),
            pl.BlockSpec((_NS, 1, _P), lambda b: (b, 0, 0)),
            full((4, _HID, _HID)), full((4, _HID, _HID)), full((4, _HID)),
            full((_HID, _HID)), full((1, _HID)),
            full((_HID, _HID)), full((_HID, _HID)), full((1, _HID)),
            full((_HID, _HID)), full((_HID, _HID)), full((1, _HID)),
            full((_HID, 256)), full((1, 256)), full((256, 256)), full((1, 256)),
        ],
        out_specs=pl.BlockSpec((_NS, 1, 256), lambda b: (b, 0, 0)),
        out_shape=jax.ShapeDtypeStruct((_B, 1, 256), f32),
    )(kg_node_feats, A, obs3,
      wn, ws, bd, wc1, bc1, wsgp, wkgp, bp, wsg2, wkg2, b2s,
      wm1, bm1, wm2, bm2)

    return out.reshape(_B, 256)


# speedup vs baseline: 1.1304x; 1.1304x over previous
import functools

import jax
import jax.numpy as jnp
from jax.experimental import pallas as pl

_B, _N, _GH, _GW = 32, 256, 16, 16
_P = _GH * _GW
_D_EDGE, _HID = 16, 128
_WH = 32


def _fused_kernel(relt_ref, adj_ref, x0_ref, obs_ref,
                  w0t_ref, b0t_ref, w1t_ref, b1t_ref, w2t_ref, b2t_ref,
                  wn_ref, ws_ref, bd_ref,
                  wc1_ref, bc1_ref,
                  wsgp_ref, wkgp_ref, bp_ref,
                  wsg2_ref, wkg2_ref, b2s_ref,
                  wm1_ref, bm1_ref, wm2_ref, bm2_ref,
                  out_ref):
    relt = relt_ref[0]
    dn = (((2,), (1,)), ((0,), (0,)))
    h = jax.lax.dot_general(
        jnp.broadcast_to(w0t_ref[...], (_N, 4 * _WH, _D_EDGE)), relt, dn,
        preferred_element_type=jnp.float32)
    h = jnp.maximum(h + b0t_ref[...], 0.0)
    h = jax.lax.dot_general(
        jnp.broadcast_to(w1t_ref[...], (_N, 4 * _WH, 4 * _WH)), h, dn,
        preferred_element_type=jnp.float32)
    h = jnp.maximum(h + b1t_ref[...], 0.0)
    logits = jax.lax.dot_general(
        jnp.broadcast_to(w2t_ref[...], (_N, 4, 4 * _WH)), h, dn,
        preferred_element_type=jnp.float32)
    wgt = jax.nn.sigmoid(logits + b2t_ref[...])
    adj = adj_ref[0]
    A = wgt * adj[:, None, :]

    x = x0_ref[0]

    def dg(x, l):
        Al = A[:, l, :]
        deg = jnp.sum(Al, axis=1, keepdims=True) + 1e-6
        msg = (Al @ x) / deg
        return jnp.maximum(msg @ wn_ref[l] + x @ ws_ref[l] + bd_ref[l], 0.0)

    x = dg(x, 0)
    x = dg(x, 1)

    node_iota = jax.lax.broadcasted_iota(jnp.int32, (_N, _P), 0)
    ohT = (node_iota == obs_ref[0]).astype(jnp.float32)
    dtl = (((0,), (0,)), ((), ()))

    sg0 = jax.lax.dot_general(ohT, x, dtl,
                              preferred_element_type=jnp.float32)
    sg = jnp.maximum(sg0 @ wc1_ref[...] + bc1_ref[...], 0.0)

    counts = jnp.sum(ohT, axis=1, keepdims=True)
    pooled = (ohT @ sg) / (counts + 1e-6)
    x = jnp.maximum(pooled @ wsgp_ref[...] + x @ wkgp_ref[...] + bp_ref[...] + x, 0.0)

    x = dg(x, 2)
    x = dg(x, 3)

    g2 = jax.lax.dot_general(ohT, x, dtl,
                             preferred_element_type=jnp.float32)
    sg = jnp.maximum(sg @ wsg2_ref[...] + g2 @ wkg2_ref[...] + b2s_ref[...], 0.0)

    v = jnp.mean(sg, axis=0, keepdims=True)
    v = jnp.maximum(v @ wm1_ref[...] + bm1_ref[...], 0.0)
    v = jnp.maximum(v @ wm2_ref[...] + bm2_ref[...], 0.0)
    out_ref[0] = v


def kernel(kg_node_feats, kg_adj, kg_rel, obs, params):
    f32 = jnp.float32

    dgs = [params['dg1'], params['dg2'], params['dg3'], params['dg4']]
    w0t = jnp.concatenate([d['wnet_ws'][0] for d in dgs], axis=1).T
    b0t = jnp.concatenate([d['wnet_bs'][0] for d in dgs])[:, None]
    w1 = jnp.zeros((4 * _WH, 4 * _WH), f32)
    for i, d in enumerate(dgs):
        w1 = w1.at[i * _WH:(i + 1) * _WH, i * _WH:(i + 1) * _WH].set(d['wnet_ws'][1])
    w1t = w1.T
    b1t = jnp.concatenate([d['wnet_bs'][1] for d in dgs])[:, None]
    w2t = jnp.zeros((4, 4 * _WH), f32)
    for i, d in enumerate(dgs):
        w2t = w2t.at[i, i * _WH:(i + 1) * _WH].set(d['wnet_ws'][2][:, 0])
    b2t = jnp.stack([d['wnet_bs'][2][0] for d in dgs])[:, None]

    rel_t = jnp.swapaxes(kg_rel, 2, 3)

    wn = jnp.stack([d['w_nbr'] for d in dgs])
    ws = jnp.stack([d['w_self'] for d in dgs])
    bd = jnp.stack([d['b'] for d in dgs])
    sgc1, sgc2, sgkg = params['sgc1'], params['sgc2'], params['sgkg1']
    wc1 = sgc1['w_sg'] + sgc1['w_kg']
    bc1 = sgc1['b'][None, :]
    wsgp, wkgp, bp = sgkg['w_sg'], sgkg['w_kg'], sgkg['b'][None, :]
    wsg2, wkg2, b2s = sgc2['w_sg'], sgc2['w_kg'], sgc2['b'][None, :]
    wm1, bm1 = params['mlp']['ws'][0], params['mlp']['bs'][0][None, :]
    wm2, bm2 = params['mlp']['ws'][1], params['mlp']['bs'][1][None, :]

    obs3 = obs.reshape(_B, 1, _P)

    full = lambda shape: pl.BlockSpec(shape, lambda b: tuple(0 for _ in shape))
    out = pl.pallas_call(
        _fused_kernel,
        grid=(_B,),
        in_specs=[
            pl.BlockSpec((1, _N, _D_EDGE, _N), lambda b: (b, 0, 0, 0)),
            pl.BlockSpec((1, _N, _N), lambda b: (b, 0, 0)),
            pl.BlockSpec((1, _N, _HID), lambda b: (b, 0, 0)),
            pl.BlockSpec((1, 1, _P), lambda b: (b, 0, 0)),
            full((4 * _WH, _D_EDGE)), full((4 * _WH, 1)),
            full((4 * _WH, 4 * _WH)), full((4 * _WH, 1)),
            full((4, 4 * _WH)), full((4, 1)),
            full((4, _HID, _HID)), full((4, _HID, _HID)), full((4, _HID)),
            full((_HID, _HID)), full((1, _HID)),
            full((_HID, _HID)), full((_HID, _HID)), full((1, _HID)),
            full((_HID, _HID)), full((_HID, _HID)), full((1, _HID)),
            full((_HID, 256)), full((1, 256)), full((256, 256)), full((1, 256)),
        ],
        out_specs=pl.BlockSpec((1, 1, 256), lambda b: (b, 0, 0)),
        out_shape=jax.ShapeDtypeStruct((_B, 1, 256), f32),
    )(rel_t, kg_adj, kg_node_feats, obs3,
      w0t, b0t, w1t, b1t, w2t, b2t,
      wn, ws, bd, wc1, bc1, wsgp, wkgp, bp, wsg2, wkg2, b2s,
      wm1, bm1, wm2, bm2)

    return out.reshape(_B, 256)


# speedup vs baseline: 1.1314x; 1.0009x over previous
import functools

import jax
import jax.numpy as jnp
from jax.experimental import pallas as pl

_B, _N, _GH, _GW = 32, 256, 16, 16
_P = _GH * _GW
_D_EDGE, _HID = 16, 128
_WH = 32


def _fused_kernel(relt_ref, adj_ref, x0_ref, obs_ref,
                  w0t_ref, b0t_ref, w1t_ref, b1t_ref, w2t_ref, b2t_ref,
                  wn_ref, ws_ref, bd_ref,
                  wc1_ref, bc1_ref,
                  wsgp_ref, wkgp_ref, bp_ref,
                  wsg2_ref, wkg2_ref, b2s_ref,
                  wm1_ref, bm1_ref, wm2_ref, bm2_ref,
                  out_ref):
    relt = relt_ref[0]
    dn = (((2,), (1,)), ((0,), (0,)))
    h = jax.lax.dot_general(
        jnp.broadcast_to(w0t_ref[...], (_N, 4 * _WH, _D_EDGE)), relt, dn,
        preferred_element_type=jnp.float32)
    h = jnp.maximum(h + b0t_ref[...], 0.0)
    h = jax.lax.dot_general(
        jnp.broadcast_to(w1t_ref[...], (_N, 4 * _WH, 4 * _WH)), h, dn,
        preferred_element_type=jnp.float32)
    h = jnp.maximum(h + b1t_ref[...], 0.0)
    logits = jax.lax.dot_general(
        jnp.broadcast_to(w2t_ref[...], (_N, 4, 4 * _WH)), h, dn,
        preferred_element_type=jnp.float32)
    wgt = 0.5 * (jnp.tanh((logits + b2t_ref[...]) * 0.5) + 1.0)
    adj = adj_ref[0]
    A = wgt * adj[:, None, :]

    x = x0_ref[0]

    def dg(x, l):
        Al = A[:, l, :]
        deg = jnp.sum(Al, axis=1, keepdims=True) + 1e-6
        msg = (Al @ x) / deg
        return jnp.maximum(msg @ wn_ref[l] + x @ ws_ref[l] + bd_ref[l], 0.0)

    x = dg(x, 0)
    x = dg(x, 1)

    node_iota = jax.lax.broadcasted_iota(jnp.int32, (_N, _P), 0)
    ohT = (node_iota == obs_ref[0]).astype(jnp.float32)
    dtl = (((0,), (0,)), ((), ()))

    sg0 = jax.lax.dot_general(ohT, x, dtl,
                              preferred_element_type=jnp.float32)
    sg = jnp.maximum(sg0 @ wc1_ref[...] + bc1_ref[...], 0.0)

    counts = jnp.sum(ohT, axis=1, keepdims=True)
    pooled = (ohT @ sg) / (counts + 1e-6)
    x = jnp.maximum(pooled @ wsgp_ref[...] + x @ wkgp_ref[...] + bp_ref[...] + x, 0.0)

    x = dg(x, 2)
    x = dg(x, 3)

    g2 = jax.lax.dot_general(ohT, x, dtl,
                             preferred_element_type=jnp.float32)
    sg = jnp.maximum(sg @ wsg2_ref[...] + g2 @ wkg2_ref[...] + b2s_ref[...], 0.0)

    v = jnp.mean(sg, axis=0, keepdims=True)
    v = jnp.maximum(v @ wm1_ref[...] + bm1_ref[...], 0.0)
    v = jnp.maximum(v @ wm2_ref[...] + bm2_ref[...], 0.0)
    out_ref[0] = v


def kernel(kg_node_feats, kg_adj, kg_rel, obs, params):
    f32 = jnp.float32

    dgs = [params['dg1'], params['dg2'], params['dg3'], params['dg4']]
    w0t = jnp.concatenate([d['wnet_ws'][0] for d in dgs], axis=1).T
    b0t = jnp.concatenate([d['wnet_bs'][0] for d in dgs])[:, None]
    w1 = jnp.zeros((4 * _WH, 4 * _WH), f32)
    for i, d in enumerate(dgs):
        w1 = w1.at[i * _WH:(i + 1) * _WH, i * _WH:(i + 1) * _WH].set(d['wnet_ws'][1])
    w1t = w1.T
    b1t = jnp.concatenate([d['wnet_bs'][1] for d in dgs])[:, None]
    w2t = jnp.zeros((4, 4 * _WH), f32)
    for i, d in enumerate(dgs):
        w2t = w2t.at[i, i * _WH:(i + 1) * _WH].set(d['wnet_ws'][2][:, 0])
    b2t = jnp.stack([d['wnet_bs'][2][0] for d in dgs])[:, None]

    rel_t = jnp.swapaxes(kg_rel, 2, 3)

    wn = jnp.stack([d['w_nbr'] for d in dgs])
    ws = jnp.stack([d['w_self'] for d in dgs])
    bd = jnp.stack([d['b'] for d in dgs])
    sgc1, sgc2, sgkg = params['sgc1'], params['sgc2'], params['sgkg1']
    wc1 = sgc1['w_sg'] + sgc1['w_kg']
    bc1 = sgc1['b'][None, :]
    wsgp, wkgp, bp = sgkg['w_sg'], sgkg['w_kg'], sgkg['b'][None, :]
    wsg2, wkg2, b2s = sgc2['w_sg'], sgc2['w_kg'], sgc2['b'][None, :]
    wm1, bm1 = params['mlp']['ws'][0], params['mlp']['bs'][0][None, :]
    wm2, bm2 = params['mlp']['ws'][1], params['mlp']['bs'][1][None, :]

    obs3 = obs.reshape(_B, 1, _P)

    full = lambda shape: pl.BlockSpec(shape, lambda b: tuple(0 for _ in shape))
    out = pl.pallas_call(
        _fused_kernel,
        grid=(_B,),
        in_specs=[
            pl.BlockSpec((1, _N, _D_EDGE, _N), lambda b: (b, 0, 0, 0)),
            pl.BlockSpec((1, _N, _N), lambda b: (b, 0, 0)),
            pl.BlockSpec((1, _N, _HID), lambda b: (b, 0, 0)),
            pl.BlockSpec((1, 1, _P), lambda b: (b, 0, 0)),
            full((4 * _WH, _D_EDGE)), full((4 * _WH, 1)),
            full((4 * _WH, 4 * _WH)), full((4 * _WH, 1)),
            full((4, 4 * _WH)), full((4, 1)),
            full((4, _HID, _HID)), full((4, _HID, _HID)), full((4, _HID)),
            full((_HID, _HID)), full((1, _HID)),
            full((_HID, _HID)), full((_HID, _HID)), full((1, _HID)),
            full((_HID, _HID)), full((_HID, _HID)), full((1, _HID)),
            full((_HID, 256)), full((1, 256)), full((256, 256)), full((1, 256)),
        ],
        out_specs=pl.BlockSpec((1, 1, 256), lambda b: (b, 0, 0)),
        out_shape=jax.ShapeDtypeStruct((_B, 1, 256), f32),
    )(rel_t, kg_adj, kg_node_feats, obs3,
      w0t, b0t, w1t, b1t, w2t, b2t,
      wn, ws, bd, wc1, bc1, wsgp, wkgp, bp, wsg2, wkg2, b2s,
      wm1, bm1, wm2, bm2)

    return out.reshape(_B, 256)
